# baseline (speedup 1.0000x reference)
import functools

import jax
import jax.numpy as jnp
from jax.experimental import pallas as pl
from jax.experimental.pallas import tpu as pltpu


_ROW_BLOCK = {128: 8, 64: 16, 32: 32, 16: 16, 8: 8}
_C = 256


def _compute_block(x_ref, wsh_ref, bsh_ref, whead_ref, bhead_ref,
                   lg_ref, pr_ref, bx_ref, local_rb, row_off, W, Wp, RB):
    M = RB * W
    r0 = local_rb * RB
    xflat = x_ref[0, pl.ds(row_off + r0 * Wp, (RB + 2) * Wp), :]
    x3d = xflat.reshape(RB + 2, Wp, _C)
    acc = jnp.zeros((M, 512), jnp.float32)
    for kx in range(3):
        xk = x3d[:, kx:kx + W, :]
        for ky in range(3):
            xs = xk[ky:ky + RB].reshape(M, _C)
            acc = acc + jnp.dot(xs, wsh_ref[ky, kx],
                                preferred_element_type=jnp.float32)
    act = jnp.maximum(acc + bsh_ref[0], 0.0)
    head = jnp.dot(act.astype(jnp.bfloat16), whead_ref[...],
                   preferred_element_type=jnp.float32) + bhead_ref[0]
    logit = head[:, 0:6]
    logit_sw = head[:, 6:12]
    box = head[:, 12:24]
    m = jnp.maximum(logit, logit_sw)
    e = jnp.exp(logit - m)
    esw = jnp.exp(logit_sw - m)
    prob = e / (e + esw)
    lg_ref[0, :, :] = logit
    pr_ref[0, :, :] = prob
    bx_ref[0, :, :] = box


def _fused_body(x_ref, wsh_ref, bsh_ref, whead_ref, bhead_ref, *out_refs,
                cfg):
    rb = pl.program_id(1)
    for i, (H, W, RB, nb, s, Wp, row_off) in enumerate(cfg):
        @pl.when(jnp.logical_and(rb >= s, rb < s + nb))
        def _(i=i, H=H, W=W, RB=RB, s=s, Wp=Wp, row_off=row_off):
            _compute_block(x_ref, wsh_ref, bsh_ref, whead_ref, bhead_ref,
                           out_refs[3 * i], out_refs[3 * i + 1],
                           out_refs[3 * i + 2], rb - s, row_off, W, Wp, RB)


def kernel(feat_p2, feat_p3, feat_p4, feat_p5, feat_p6,
           W_share, b_share, W_cls, b_cls, W_box, b_box):
    feats = [feat_p2, feat_p3, feat_p4, feat_p5, feat_p6]
    B = feats[0].shape[0]
    wsh = W_share.astype(jnp.bfloat16)
    bsh = b_share.astype(jnp.float32).reshape(1, 512)
    wcls = W_cls.reshape(512, 6)
    perm = jnp.array([1, 0, 3, 2, 5, 4], dtype=jnp.int32)
    whead = jnp.concatenate(
        [wcls, wcls[:, perm], W_box.reshape(512, 12)], axis=1
    ).astype(jnp.bfloat16)
    bhead = jnp.concatenate(
        [b_cls, b_cls[perm], b_box]
    ).astype(jnp.float32).reshape(1, 24)

    cfg = []
    flat_parts = []
    s = 0
    row_off = 0
    for x in feats:
        _, H, W, _ = x.shape
        RB = _ROW_BLOCK[H]
        nb = H // RB
        Wp = (W + 2 + 7) // 8 * 8
        xp = jnp.pad(x.astype(jnp.bfloat16),
                     ((0, 0), (1, 1), (1, Wp - W - 1), (0, 0)))
        flat_parts.append(xp.reshape(B, (H + 2) * Wp, _C))
        cfg.append((H, W, RB, nb, s, Wp, row_off))
        s += nb
        row_off += (H + 2) * Wp
    n_steps = s
    xcat = jnp.concatenate(flat_parts, axis=1)
    n_rows = xcat.shape[1]

    in_specs = [
        pl.BlockSpec((1, n_rows, _C), lambda b, rb: (b, 0, 0)),
        pl.BlockSpec((3, 3, _C, 512), lambda b, rb: (0, 0, 0, 0)),
        pl.BlockSpec((1, 512), lambda b, rb: (0, 0)),
        pl.BlockSpec((512, 24), lambda b, rb: (0, 0)),
        pl.BlockSpec((1, 24), lambda b, rb: (0, 0)),
    ]
    out_shape = []
    out_specs = []
    for (H, W, RB, nb, st, Wp, ro) in cfg:
        M = RB * W
        for ch in (6, 6, 12):
            out_shape.append(jax.ShapeDtypeStruct((B, H * W, ch), jnp.float32))
            out_specs.append(pl.BlockSpec(
                (1, M, ch),
                functools.partial(
                    lambda b, rb, st=st, nb=nb: (b, jnp.clip(rb - st, 0, nb - 1), 0))))

    f = pl.pallas_call(
        functools.partial(_fused_body, cfg=cfg),
        grid=(B, n_steps),
        in_specs=in_specs,
        out_specs=out_specs,
        out_shape=out_shape,
        compiler_params=pltpu.CompilerParams(
            dimension_semantics=("parallel", "arbitrary")))
    outs = f(xcat, wsh, bsh, whead, bhead)

    logits_all, probs_all, boxes_all = [], [], []
    for i, x in enumerate(feats):
        _, H, W, _ = x.shape
        lg, pr, bx = outs[3 * i], outs[3 * i + 1], outs[3 * i + 2]
        logits_all.append(lg.reshape(B, H * W * 3, 2))
        probs_all.append(pr.reshape(B, H * W * 3, 2))
        boxes_all.append(bx.reshape(B, H * W * 3, 4))
    class_logit = jnp.concatenate(logits_all, axis=1)
    class_prob = jnp.concatenate(probs_all, axis=1)
    box_pred = jnp.concatenate(boxes_all, axis=1)
    return (class_logit, class_prob, box_pred)
